# Initial kernel scaffold; baseline (speedup 1.0000x reference)
import jax
import jax.numpy as jnp
from jax import lax
from jax.experimental import pallas as pl
from jax.experimental.pallas import tpu as pltpu
from jax.experimental.pallas import tpu_sc as plsc

NC = 2
NS = 16
NW = NC * NS
CH = 128
DEGW = 16


def _make_sc_agg(n_nodes, d, n_edges, with_deg):
    nch = n_edges // CH
    rpt = n_nodes // NS
    mesh = plsc.VectorSubcoreMesh(
        core_axis_name="c", subcore_axis_name="s", num_cores=NC,
        num_subcores=NS)

    out_type = [jax.ShapeDtypeStruct((NC, n_nodes, d), jnp.float32)]
    scratch = [
        pltpu.VMEM((CH,), jnp.int32),
        pltpu.VMEM((CH,), jnp.int32),
        pltpu.VMEM((CH, d), jnp.float32),
        pltpu.VMEM_SHARED((n_nodes, d), jnp.float32),
        pltpu.SemaphoreType.DMA,
    ]
    if with_deg:
        out_type.append(jax.ShapeDtypeStruct((NC, n_nodes, DEGW), jnp.float32))
        scratch += [
            pltpu.VMEM((CH, DEGW), jnp.float32),
            pltpu.VMEM_SHARED((n_nodes, DEGW), jnp.float32),
        ]

    def body(h_hbm, src_hbm, dst_hbm, zagg_hbm, zdeg_hbm, ones_hbm, *rest):
        if with_deg:
            (agg_out, deg_out, srcv, dstv, rows, agg_sh, sem, onesv,
             deg_sh) = rest
        else:
            agg_out, srcv, dstv, rows, agg_sh, sem = rest
        cid = lax.axis_index("c")
        sid = lax.axis_index("s")
        wid = sid * NC + cid
        r0 = sid * rpt

        pltpu.sync_copy(zagg_hbm.at[pl.ds(r0, rpt)], agg_sh.at[pl.ds(r0, rpt)])
        if with_deg:
            pltpu.sync_copy(zdeg_hbm.at[pl.ds(r0, rpt)],
                            deg_sh.at[pl.ds(r0, rpt)])
            pltpu.sync_copy(ones_hbm, onesv)
        plsc.subcore_barrier()

        n_g = (nch - wid + NW - 1) // NW

        def chunk(g, carry):
            base = (wid + g * NW) * CH
            pltpu.sync_copy(src_hbm.at[pl.ds(base, CH)], srcv)
            pltpu.sync_copy(dst_hbm.at[pl.ds(base, CH)], dstv)
            pltpu.async_copy(h_hbm.at[srcv], rows, sem).wait()
            pltpu.sync_copy(rows, agg_sh.at[dstv], add=True)
            if with_deg:
                pltpu.sync_copy(onesv, deg_sh.at[dstv], add=True)
            return carry

        lax.fori_loop(0, n_g, chunk, 0)
        plsc.subcore_barrier()

        pltpu.sync_copy(agg_sh.at[pl.ds(r0, rpt)],
                        agg_out.at[cid, pl.ds(r0, rpt)])
        if with_deg:
            pltpu.sync_copy(deg_sh.at[pl.ds(r0, rpt)],
                            deg_out.at[cid, pl.ds(r0, rpt)])

    return pl.kernel(body, out_type=tuple(out_type), mesh=mesh,
                     scratch_types=scratch)


def _leaky(x):
    return jnp.where(x >= 0, x, 0.1 * x)


def _encode_body(x_ref, w_ref, b_ref, o_ref):
    h = jnp.dot(x_ref[...], w_ref[...],
                preferred_element_type=jnp.float32) + b_ref[...]
    o_ref[...] = _leaky(h)


def _sage_body(h_ref, agg_ref, deg_ref, wl_ref, bl_ref, wr_ref, o_ref):
    agg = agg_ref[0] + agg_ref[1]
    deg = deg_ref[0, :, 0:1] + deg_ref[1, :, 0:1]
    mean = agg / jnp.maximum(deg, 1.0)
    out = (jnp.dot(mean, wl_ref[...], preferred_element_type=jnp.float32)
           + bl_ref[...]
           + jnp.dot(h_ref[...], wr_ref[...],
                     preferred_element_type=jnp.float32))
    o_ref[...] = _leaky(out)


def _tc_encode(x, w, b, br):
    n, d = x.shape
    return pl.pallas_call(
        _encode_body,
        grid=(n // br,),
        in_specs=[
            pl.BlockSpec((br, d), lambda i: (i, 0)),
            pl.BlockSpec((d, d), lambda i: (0, 0)),
            pl.BlockSpec((1, d), lambda i: (0, 0)),
        ],
        out_specs=pl.BlockSpec((br, d), lambda i: (i, 0)),
        out_shape=jax.ShapeDtypeStruct((n, d), jnp.float32),
    )(x, w, b.reshape(1, d))


def _tc_sage(h, agg_parts, deg_parts, wl, bl, wr, br):
    n, d = h.shape
    return pl.pallas_call(
        _sage_body,
        grid=(n // br,),
        in_specs=[
            pl.BlockSpec((br, d), lambda i: (i, 0)),
            pl.BlockSpec((NC, br, d), lambda i: (0, i, 0)),
            pl.BlockSpec((NC, br, DEGW), lambda i: (0, i, 0)),
            pl.BlockSpec((d, d), lambda i: (0, 0)),
            pl.BlockSpec((1, d), lambda i: (0, 0)),
            pl.BlockSpec((d, d), lambda i: (0, 0)),
        ],
        out_specs=pl.BlockSpec((br, d), lambda i: (i, 0)),
        out_shape=jax.ShapeDtypeStruct((n, d), jnp.float32),
    )(h, agg_parts, deg_parts, wl, bl.reshape(1, d), wr)


def kernel(x, edge_index, W_enc, b_enc, Wl0, bl0, Wr0, Wl1, bl1, Wr1):
    n, d = x.shape
    e = edge_index.shape[1]
    src = edge_index[0].astype(jnp.int32)
    dst = edge_index[1].astype(jnp.int32)

    zagg = jnp.zeros((n, d), jnp.float32)
    zdeg = jnp.zeros((n, DEGW), jnp.float32)
    ones16 = jnp.ones((CH, DEGW), jnp.float32)

    sc_agg_deg = _make_sc_agg(n, d, e, with_deg=True)
    sc_agg = _make_sc_agg(n, d, e, with_deg=False)

    br = 1000
    h0 = _tc_encode(x, W_enc, b_enc, br)
    agg0, deg = sc_agg_deg(h0, src, dst, zagg, zdeg, ones16)
    h1 = _tc_sage(h0, agg0, deg, Wl0, bl0, Wr0, br)
    agg1 = sc_agg(h1, src, dst, zagg, zdeg, ones16)
    if isinstance(agg1, (tuple, list)):
        agg1 = agg1[0]
    h2 = _tc_sage(h1, agg1, deg, Wl1, bl1, Wr1, br)
    return h2


# speedup vs baseline: 3.9722x; 3.9722x over previous
import jax
import jax.numpy as jnp
from jax import lax
from jax.experimental import pallas as pl
from jax.experimental.pallas import tpu as pltpu
from jax.experimental.pallas import tpu_sc as plsc

NC = 2
NS = 16
NW = NC * NS
CH = 64


def _edge_partition(n_edges):
    nch = n_edges // CH
    return nch // NW, nch - (nch // NW) * NW


def _stage_zero(zsrc, buf, sh, r0, rpt):
    for k in range(0, rpt, CH):
        sz = min(CH, rpt - k)
        pltpu.sync_copy(zsrc.at[pl.ds(r0 + k, sz)], buf.at[pl.ds(0, sz)])
        pltpu.sync_copy(buf.at[pl.ds(0, sz)], sh.at[pl.ds(r0 + k, sz)])


def _stage_out(sh, buf, out, cid, r0, rpt):
    for k in range(0, rpt, CH):
        sz = min(CH, rpt - k)
        pltpu.sync_copy(sh.at[pl.ds(r0 + k, sz)], buf.at[pl.ds(0, sz)])
        pltpu.sync_copy(buf.at[pl.ds(0, sz)],
                        out.at[cid, pl.ds(r0 + k, sz)])


def _make_sc_agg(n_nodes, d, n_edges):
    rpt = n_nodes // NS
    n_full, n_tail = _edge_partition(n_edges)
    mesh = plsc.VectorSubcoreMesh(core_axis_name="c", subcore_axis_name="s",
                                  num_cores=NC, num_subcores=NS)

    def body(h_hbm, src_hbm, dst_hbm, zero_hbm, agg_out,
             srcv, dstv, rows, agg_sh, sem):
        cid = lax.axis_index("c")
        sid = lax.axis_index("s")
        wid = sid * NC + cid
        r0 = sid * rpt
        _stage_zero(zero_hbm, rows, agg_sh, r0, rpt)
        plsc.subcore_barrier()

        def do_chunk(chunk_idx):
            base = chunk_idx * CH
            pltpu.sync_copy(src_hbm.at[pl.ds(base, CH)], srcv)
            pltpu.sync_copy(dst_hbm.at[pl.ds(base, CH)], dstv)
            pltpu.async_copy(h_hbm.at[srcv], rows, sem).wait()
            pltpu.sync_copy(rows, agg_sh.at[dstv], add=True)

        def chunk(g, carry):
            do_chunk(wid + g * NW)
            return carry

        lax.fori_loop(0, n_full, chunk, 0)
        if n_tail:
            @pl.when(wid < n_tail)
            def _():
                do_chunk(n_full * NW + wid)
        plsc.subcore_barrier()
        _stage_out(agg_sh, rows, agg_out, cid, r0, rpt)

    return pl.kernel(
        body,
        out_type=jax.ShapeDtypeStruct((NC, n_nodes, d), jnp.float32),
        mesh=mesh,
        scratch_types=[
            pltpu.VMEM((CH,), jnp.int32),
            pltpu.VMEM((CH,), jnp.int32),
            pltpu.VMEM((CH, d), jnp.float32),
            pltpu.VMEM_SHARED((n_nodes, d), jnp.float32),
            pltpu.SemaphoreType.DMA,
        ])


def _make_sc_deg(n_nodes, d, n_edges):
    rpt = n_nodes // NS
    n_full, n_tail = _edge_partition(n_edges)
    mesh = plsc.VectorSubcoreMesh(core_axis_name="c", subcore_axis_name="s",
                                  num_cores=NC, num_subcores=NS)

    def body(dst_hbm, zero_hbm, ones_hbm, deg_out, dstv, ones, buf, deg_sh):
        cid = lax.axis_index("c")
        sid = lax.axis_index("s")
        wid = sid * NC + cid
        r0 = sid * rpt
        _stage_zero(zero_hbm, buf, deg_sh, r0, rpt)
        pltpu.sync_copy(ones_hbm, ones)
        plsc.subcore_barrier()

        def do_chunk(chunk_idx):
            base = chunk_idx * CH
            pltpu.sync_copy(dst_hbm.at[pl.ds(base, CH)], dstv)
            pltpu.sync_copy(ones, deg_sh.at[dstv], add=True)

        def chunk(g, carry):
            do_chunk(wid + g * NW)
            return carry

        lax.fori_loop(0, n_full, chunk, 0)
        if n_tail:
            @pl.when(wid < n_tail)
            def _():
                do_chunk(n_full * NW + wid)
        plsc.subcore_barrier()
        _stage_out(deg_sh, buf, deg_out, cid, r0, rpt)

    return pl.kernel(
        body,
        out_type=jax.ShapeDtypeStruct((NC, n_nodes, d), jnp.float32),
        mesh=mesh,
        scratch_types=[
            pltpu.VMEM((CH,), jnp.int32),
            pltpu.VMEM((CH, d), jnp.float32),
            pltpu.VMEM((CH, d), jnp.float32),
            pltpu.VMEM_SHARED((n_nodes, d), jnp.float32),
        ])


def _leaky(x):
    return jnp.where(x >= 0, x, 0.1 * x)


def _encode_body(x_ref, w_ref, b_ref, o_ref):
    h = jnp.dot(x_ref[...], w_ref[...],
                preferred_element_type=jnp.float32) + b_ref[...]
    o_ref[...] = _leaky(h)


def _sage_body(h_ref, agg_ref, deg_ref, wl_ref, bl_ref, wr_ref, o_ref):
    agg = agg_ref[0] + agg_ref[1]
    deg = deg_ref[0, :, 0:1] + deg_ref[1, :, 0:1]
    mean = agg / jnp.maximum(deg, 1.0)
    out = (jnp.dot(mean, wl_ref[...], preferred_element_type=jnp.float32)
           + bl_ref[...]
           + jnp.dot(h_ref[...], wr_ref[...],
                     preferred_element_type=jnp.float32))
    o_ref[...] = _leaky(out)


def _tc_encode(x, w, b, br):
    n, d = x.shape
    return pl.pallas_call(
        _encode_body,
        grid=(n // br,),
        in_specs=[
            pl.BlockSpec((br, d), lambda i: (i, 0)),
            pl.BlockSpec((d, d), lambda i: (0, 0)),
            pl.BlockSpec((1, d), lambda i: (0, 0)),
        ],
        out_specs=pl.BlockSpec((br, d), lambda i: (i, 0)),
        out_shape=jax.ShapeDtypeStruct((n, d), jnp.float32),
    )(x, w, b.reshape(1, d))


def _tc_sage(h, agg_parts, deg_parts, wl, bl, wr, br):
    n, d = h.shape
    return pl.pallas_call(
        _sage_body,
        grid=(n // br,),
        in_specs=[
            pl.BlockSpec((br, d), lambda i: (i, 0)),
            pl.BlockSpec((NC, br, d), lambda i: (0, i, 0)),
            pl.BlockSpec((NC, br, d), lambda i: (0, i, 0)),
            pl.BlockSpec((d, d), lambda i: (0, 0)),
            pl.BlockSpec((1, d), lambda i: (0, 0)),
            pl.BlockSpec((d, d), lambda i: (0, 0)),
        ],
        out_specs=pl.BlockSpec((br, d), lambda i: (i, 0)),
        out_shape=jax.ShapeDtypeStruct((n, d), jnp.float32),
    )(h, agg_parts, deg_parts, wl, bl.reshape(1, d), wr)


def kernel(x, edge_index, W_enc, b_enc, Wl0, bl0, Wr0, Wl1, bl1, Wr1):
    n, d = x.shape
    e = edge_index.shape[1]
    src = edge_index[0].astype(jnp.int32)
    dst = edge_index[1].astype(jnp.int32)

    np_ = -(-n // (NS * 8)) * (NS * 8)
    xp = jnp.pad(x, ((0, np_ - n), (0, 0)))

    zeros = jnp.zeros((np_, d), jnp.float32)
    ones = jnp.ones((CH, d), jnp.float32)

    sc_agg = _make_sc_agg(np_, d, e)
    sc_deg = _make_sc_deg(np_, d, e)

    br = np_ // 16
    deg = sc_deg(dst, zeros, ones)
    h0 = _tc_encode(xp, W_enc, b_enc, br)
    agg0 = sc_agg(h0, src, dst, zeros)
    h1 = _tc_sage(h0, agg0, deg, Wl0, bl0, Wr0, br)
    agg1 = sc_agg(h1, src, dst, zeros)
    h2 = _tc_sage(h1, agg1, deg, Wl1, bl1, Wr1, br)
    return h2[:n]
